# Initial kernel scaffold; baseline (speedup 1.0000x reference)
import jax
import jax.numpy as jnp
from jax.experimental import pallas as pl


def kernel(x):
    raise NotImplementedError("write your pallas kernel here")


# speedup vs baseline: 1.1794x; 1.1794x over previous
import functools

import jax
import jax.numpy as jnp
from jax import lax
from jax.experimental import pallas as pl
from jax.experimental.pallas import tpu as pltpu
from jax.experimental.pallas import tpu_sc as plsc

C = 1000


def kernel(x):
    B, S = x.shape
    N = B * S
    xf = x.reshape(N).astype(jnp.int32)

    info = plsc.get_sparse_core_info()
    NC, NS, L = info.num_cores, info.num_subcores, info.num_lanes
    NW = NC * NS
    RPW = N // NW
    G = 32
    NG = RPW // G

    mesh = plsc.VectorSubcoreMesh(core_axis_name="c", subcore_axis_name="s")

    @functools.partial(
        pl.kernel,
        mesh=mesh,
        compiler_params=pltpu.CompilerParams(needs_layout_passes=False),
        out_type=jax.ShapeDtypeStruct((N * C,), jnp.float32),
        scratch_types=[
            pltpu.VMEM((RPW,), jnp.int32),
            pltpu.VMEM((G * C,), jnp.float32),
        ],
    )
    def k(x_hbm, out_hbm, idx_v, buf_v):
        wid = lax.axis_index("s") * NC + lax.axis_index("c")
        base = wid * RPW
        pltpu.sync_copy(x_hbm.at[pl.ds(base, RPW)], idx_v)

        zeros = jnp.zeros((L,), jnp.float32)
        ones = jnp.ones((L,), jnp.float32)
        lanec = lax.iota(jnp.int32, L) * C

        def zbody(i, carry):
            buf_v[pl.ds(i * L, L)] = zeros
            return carry

        lax.fori_loop(0, (G * C) // L, zbody, 0)

        def gbody(g, carry):
            for r in range(G // L):
                idx16 = idx_v[pl.ds(g * G + r * L, L)]
                pos = idx16 + lanec + (r * L * C)
                plsc.store_scatter(buf_v, [pos], ones)
            pltpu.sync_copy(buf_v, out_hbm.at[pl.ds((base + g * G) * C, G * C)])
            for r in range(G // L):
                idx16 = idx_v[pl.ds(g * G + r * L, L)]
                pos = idx16 + lanec + (r * L * C)
                plsc.store_scatter(buf_v, [pos], zeros)
            return carry

        lax.fori_loop(0, NG, gbody, 0)

    out = k(xf)
    return out.reshape(B, S, C)
